# Initial kernel scaffold; baseline (speedup 1.0000x reference)
import jax
import jax.numpy as jnp
from jax.experimental import pallas as pl


def kernel(x, W_main, W_aux):
    raise NotImplementedError("write your pallas kernel here")


# speedup vs baseline: 5.9603x; 5.9603x over previous
import functools

import jax
import jax.numpy as jnp
from jax import lax
from jax.experimental import pallas as pl
from jax.experimental.pallas import tpu as pltpu
from jax.experimental.pallas import tpu_sc as plsc

DIM = 64
NC = 2
NS = 16
NW = NC * NS


@functools.lru_cache(maxsize=None)
def _make_gather(n_rows, dim, chunk):
    assert n_rows % (NW * chunk) == 0
    b_per_w = n_rows // NW
    n_chunks = b_per_w // chunk
    mesh = plsc.VectorSubcoreMesh(core_axis_name="c", subcore_axis_name="s")

    @functools.partial(
        pl.kernel,
        mesh=mesh,
        out_type=jax.ShapeDtypeStruct((n_rows, dim), jnp.float32),
        scratch_types=[
            pltpu.VMEM((b_per_w,), jnp.int32),
            pltpu.VMEM((2, chunk, dim), jnp.float32),
            pltpu.SemaphoreType.DMA,
            pltpu.SemaphoreType.DMA,
        ],
        compiler_params=pltpu.CompilerParams(use_tc_tiling_on_sc=False),
    )
    def gather_kernel(table_hbm, idx_hbm, out_hbm, idx_v, rows_v, gsem0, gsem1):
        wid = lax.axis_index("s") * NC + lax.axis_index("c")
        base = wid * b_per_w
        pltpu.sync_copy(idx_hbm.at[pl.ds(base, b_per_w)], idx_v)

        def gstart(i, slot, sem):
            pltpu.async_copy(
                table_hbm.at[idx_v.at[pl.ds(i * chunk, chunk)]],
                rows_v.at[slot],
                sem,
            )

        def gwait(slot, sem):
            pltpu.make_async_copy(
                table_hbm.at[idx_v.at[pl.ds(0, chunk)]],
                rows_v.at[slot],
                sem,
            ).wait()

        def store(i, slot):
            pltpu.sync_copy(
                rows_v.at[slot],
                out_hbm.at[pl.ds(base + i * chunk, chunk)],
            )

        gstart(0, 0, gsem0)

        def body(i, carry):
            slot = lax.rem(i, 2)

            @pl.when(i + 1 < n_chunks)
            def _():
                nslot = lax.rem(i + 1, 2)

                @pl.when(nslot == 0)
                def _():
                    gstart(i + 1, 0, gsem0)

                @pl.when(nslot == 1)
                def _():
                    gstart(i + 1, 1, gsem1)

            @pl.when(slot == 0)
            def _():
                gwait(0, gsem0)

            @pl.when(slot == 1)
            def _():
                gwait(1, gsem1)

            store(i, slot)
            return carry

        lax.fori_loop(0, n_chunks, body, 0, unroll=2)

    return gather_kernel


def kernel(x, W_main, W_aux):
    table = jnp.concatenate([W_main, W_aux], axis=0)
    batch, hist = x.shape
    n_rows = batch * hist
    idx = x.reshape(n_rows)
    out = _make_gather(n_rows, DIM, 512)(table, idx)
    return out.reshape(batch, hist, DIM)
